# baseline (speedup 1.0000x reference)
import jax
import jax.numpy as jnp
from jax.experimental import pallas as pl
from jax.experimental.pallas import tpu as pltpu

N = 10000
D = 128
E = 320000

_ROWS = 1000


def _scale_kernel(g_ref, h_ref, o_ref):
    o_ref[...] = g_ref[...] * h_ref[...]


def _mm_kernel(a_ref, w_ref, o_ref):
    o_ref[...] = jnp.dot(a_ref[...], w_ref[...],
                         preferred_element_type=jnp.float32)


def _matmul(a, w):
    return pl.pallas_call(
        _mm_kernel,
        out_shape=jax.ShapeDtypeStruct((N, D), jnp.float32),
        grid=(N // _ROWS,),
        in_specs=[
            pl.BlockSpec((_ROWS, D), lambda i: (i, 0)),
            pl.BlockSpec((D, D), lambda i: (0, 0)),
        ],
        out_specs=pl.BlockSpec((_ROWS, D), lambda i: (i, 0)),
    )(a, w)


def kernel(x, edge_index, W1, b1, Wp, bp):
    src, dst = edge_index[0], edge_index[1]
    loop = jnp.arange(N, dtype=src.dtype)
    s = jnp.concatenate([src, loop])
    d = jnp.concatenate([dst, loop])
    order_e = jnp.argsort(d, stable=True)
    s2, d2 = s[order_e], d[order_e]
    deg = jnp.zeros((N,), x.dtype).at[d].add(1.0)
    dinv = jnp.where(deg > 0, 1.0 / jnp.sqrt(deg), 0.0)
    norm2 = dinv[s2] * dinv[d2]

    def conv(h):
        hw = _matmul(h, W1)
        msg = hw[s2] * norm2[:, None]
        return jnp.zeros((N, D), x.dtype).at[d2].add(msg) + b1

    h = jax.nn.relu(conv(x))
    h1 = conv(h)
    g = h1 @ Wp + bp
    order = jnp.argsort(g[:, 0])
    gs = g[order]
    h1p = h1[order]
    scaled = pl.pallas_call(
        _scale_kernel,
        out_shape=jax.ShapeDtypeStruct((N, D), x.dtype),
        grid=(N // _ROWS,),
        in_specs=[
            pl.BlockSpec((_ROWS, 1), lambda i: (i, 0)),
            pl.BlockSpec((_ROWS, D), lambda i: (i, 0)),
        ],
        out_specs=pl.BlockSpec((_ROWS, D), lambda i: (i, 0)),
    )(gs, h1p)
    return jnp.transpose(scaled, (1, 0))[None, :, :]


# speedup vs baseline: 1.5657x; 1.5657x over previous
import functools

import jax
import jax.numpy as jnp
import numpy as np
from jax import lax
from jax.experimental import pallas as pl
from jax.experimental.pallas import tpu as pltpu
from jax.experimental.pallas import tpu_sc as plsc

N = 10000
D = 128
E = 320000
ETOT = E + N
CHUNK = 128
NCHIP = 32
PAD = ((ETOT + CHUNK - 1) // CHUNK + 1) * CHUNK

_BOUNDS = np.array([8 * ((w * N) // (NCHIP * 8)) for w in range(NCHIP + 1)],
                   dtype=np.int32)
_NB_MAX = int(np.max(np.diff(_BOUNDS)))

_ROWS = 1000


def _scale_kernel(g_ref, h_ref, o_ref):
    o_ref[...] = g_ref[...] * h_ref[...]


def _mm_kernel(a_ref, w_ref, o_ref):
    o_ref[...] = jnp.dot(a_ref[...], w_ref[...],
                         preferred_element_type=jnp.float32)


def _matmul(a, w):
    return pl.pallas_call(
        _mm_kernel,
        out_shape=jax.ShapeDtypeStruct((N, D), jnp.float32),
        grid=(N // _ROWS,),
        in_specs=[
            pl.BlockSpec((_ROWS, D), lambda i: (i, 0)),
            pl.BlockSpec((D, D), lambda i: (0, 0)),
        ],
        out_specs=pl.BlockSpec((_ROWS, D), lambda i: (i, 0)),
    )(a, w)


def _agg_body(src_hbm, dst_hbm, nrm_hbm, hw_hbm, ptr_hbm, out_hbm,
              ptr_v, idx_v, dstc_v, nrmc_v, rows_v, out_v, sem):
    c = lax.axis_index("c")
    s = lax.axis_index("s")
    wid = c * 16 + s
    pltpu.sync_copy(ptr_hbm, ptr_v.at[pl.ds(0, 40)])
    pv = ptr_v[pl.ds(wid, 16)]
    e0 = pv[0]
    e1 = pv[1]
    n0 = 8 * ((wid * N) // (NCHIP * 8))
    n1 = 8 * (((wid + 1) * N) // (NCHIP * 8))
    e0a = (e0 // 8) * 8
    nch = (e1 - e0a + CHUNK - 1) // CHUNK

    def chunk_body(g, carry):
        accs, prev_d = carry
        base = e0a + g * CHUNK
        pltpu.sync_copy(src_hbm.at[pl.ds(base, CHUNK)], idx_v)
        pltpu.sync_copy(dst_hbm.at[pl.ds(base, CHUNK)], dstc_v.at[pl.ds(0, CHUNK)])
        pltpu.sync_copy(nrm_hbm.at[pl.ds(base, CHUNK)], nrmc_v.at[pl.ds(0, CHUNK)])
        pltpu.async_copy(hw_hbm.at[idx_v], rows_v, sem).wait()
        k0 = jnp.maximum(e0 - base, 0)
        k1 = jnp.minimum(e1 - base, CHUNK)

        def edge_body(k, ec):
            eaccs, eprev = ec
            dval = dstc_v[pl.ds(k, 16)][0]
            nrm = nrmc_v[pl.ds(k, 16)][0]
            sel = jnp.where(dval == eprev, jnp.float32(1.0), jnp.float32(0.0))
            r = dval - n0
            new = []
            for j in range(8):
                row = rows_v[k, pl.ds(j * 16, 16)]
                tmp = row * nrm
                a = eaccs[j] * sel + tmp
                out_v[r, pl.ds(j * 16, 16)] = a
                new.append(a)
            return tuple(new), dval

        return lax.fori_loop(k0, k1, edge_body, (accs, prev_d))

    zero = jnp.zeros((16,), jnp.float32)
    accs0 = tuple(zero for _ in range(8))
    lax.fori_loop(0, nch, chunk_body, (accs0, jnp.int32(-1)))
    pltpu.sync_copy(out_v.at[pl.ds(0, 312)], out_hbm.at[pl.ds(n0, 312)])

    @pl.when(n1 - n0 == 320)
    def _():
        pltpu.sync_copy(out_v.at[pl.ds(312, 8)], out_hbm.at[pl.ds(n0 + 312, 8)])


_agg_call = functools.partial(
    pl.kernel,
    out_type=jax.ShapeDtypeStruct((N, D), jnp.float32),
    mesh=plsc.VectorSubcoreMesh(core_axis_name="c", subcore_axis_name="s",
                                num_cores=2, num_subcores=16),
    scratch_types=[
        pltpu.VMEM((56,), jnp.int32),
        pltpu.VMEM((CHUNK,), jnp.int32),
        pltpu.VMEM((CHUNK + 16,), jnp.int32),
        pltpu.VMEM((CHUNK + 16,), jnp.float32),
        pltpu.VMEM((CHUNK, D), jnp.float32),
        pltpu.VMEM((_NB_MAX, D), jnp.float32),
        pltpu.SemaphoreType.DMA,
    ],
)(_agg_body)


def kernel(x, edge_index, W1, b1, Wp, bp):
    src, dst = edge_index[0], edge_index[1]
    loop = jnp.arange(N, dtype=src.dtype)
    s = jnp.concatenate([src, loop])
    d = jnp.concatenate([dst, loop])
    order_e = jnp.argsort(d, stable=True)
    s2, d2 = s[order_e], d[order_e]
    deg = jnp.zeros((N,), x.dtype).at[d].add(1.0)
    dinv = jnp.where(deg > 0, 1.0 / jnp.sqrt(deg), 0.0)
    norm2 = dinv[s2] * dinv[d2]

    pad_i = jnp.full((PAD - ETOT,), N, dtype=jnp.int32)
    pad_f = jnp.zeros((PAD - ETOT,), dtype=jnp.float32)
    s2p = jnp.concatenate([s2, jnp.zeros((PAD - ETOT,), jnp.int32)])
    d2p = jnp.concatenate([d2, pad_i])
    n2p = jnp.concatenate([norm2, pad_f])
    ptr = jnp.searchsorted(d2, jnp.asarray(_BOUNDS)).astype(jnp.int32)
    ptr = jnp.concatenate([ptr, jnp.zeros((40 - NCHIP - 1,), jnp.int32)])

    def conv(h):
        hw = _matmul(h, W1)
        return _agg_call(s2p, d2p, n2p, hw, ptr) + b1

    h = jax.nn.relu(conv(x))
    h1 = conv(h)
    g = h1 @ Wp + bp
    order = jnp.argsort(g[:, 0])
    gs = g[order]
    h1p = h1[order]
    scaled = pl.pallas_call(
        _scale_kernel,
        out_shape=jax.ShapeDtypeStruct((N, D), x.dtype),
        grid=(N // _ROWS,),
        in_specs=[
            pl.BlockSpec((_ROWS, 1), lambda i: (i, 0)),
            pl.BlockSpec((_ROWS, D), lambda i: (i, 0)),
        ],
        out_specs=pl.BlockSpec((_ROWS, D), lambda i: (i, 0)),
    )(gs, h1p)
    return jnp.transpose(scaled, (1, 0))[None, :, :]


# speedup vs baseline: 4.4050x; 2.8134x over previous
import functools

import jax
import jax.numpy as jnp
import numpy as np
from jax import lax
from jax.experimental import pallas as pl
from jax.experimental.pallas import tpu as pltpu
from jax.experimental.pallas import tpu_sc as plsc

N = 10000
D = 128
E = 320000
NCHIP = 32

PCHUNK = 2048
NPCH = E // PCHUNK
NPCH = (E + PCHUNK - 1) // PCHUNK
EPAD = (NPCH + 5) * PCHUNK
CAP = 12288

ACHUNK = 128

_BOUNDS = np.array([8 * ((w * N) // (NCHIP * 8)) for w in range(NCHIP + 1)],
                   dtype=np.int32)
_NB_MAX = int(np.max(np.diff(_BOUNDS)))

_ROWS = 1000

_MESH = plsc.VectorSubcoreMesh(core_axis_name="c", subcore_axis_name="s",
                               num_cores=2, num_subcores=16)


def _scale_kernel(g_ref, h_ref, o_ref):
    o_ref[...] = g_ref[...] * h_ref[...]


def _mm_kernel(a_ref, w_ref, o_ref):
    o_ref[...] = jnp.dot(a_ref[...], w_ref[...],
                         preferred_element_type=jnp.float32)


def _matmul(a, w):
    return pl.pallas_call(
        _mm_kernel,
        out_shape=jax.ShapeDtypeStruct((N, D), jnp.float32),
        grid=(N // _ROWS,),
        in_specs=[
            pl.BlockSpec((_ROWS, D), lambda i: (i, 0)),
            pl.BlockSpec((D, D), lambda i: (0, 0)),
        ],
        out_specs=pl.BlockSpec((_ROWS, D), lambda i: (i, 0)),
    )(a, w)


def _wid_bounds():
    c = lax.axis_index("c")
    s = lax.axis_index("s")
    wid = c * 16 + s
    n0 = 8 * ((wid * N) // (NCHIP * 8))
    n1 = 8 * (((wid + 1) * N) // (NCHIP * 8))
    return wid, n0, n1


def _agg_body(tsrc_hbm, tdst_hbm, tcnt_hbm, dinv_hbm, hw_hbm, out_hbm,
              idx_v, dstc_v, nrm_v, dinv_v, rows_v, out_v, cbuf, sem):
    wid, n0, n1 = _wid_bounds()
    pltpu.sync_copy(tcnt_hbm.at[pl.ds(wid * 16, 16)], cbuf)
    cnt = cbuf[pl.ds(0, 16)][0]
    pltpu.sync_copy(dinv_hbm, dinv_v.at[pl.ds(0, N)])

    zero = jnp.zeros((16,), jnp.float32)

    def zrow(r, _):
        for j in range(8):
            out_v[r, pl.ds(j * 16, 16)] = zero
        return 0

    lax.fori_loop(0, n1 - n0, zrow, 0)

    nch = (cnt + ACHUNK - 1) // ACHUNK

    def chunk_body(g, _):
        base = wid * CAP + g * ACHUNK
        pltpu.sync_copy(tsrc_hbm.at[pl.ds(base, ACHUNK)], idx_v)
        pltpu.sync_copy(tdst_hbm.at[pl.ds(base, ACHUNK)],
                        dstc_v.at[pl.ds(0, ACHUNK)])
        pltpu.async_copy(hw_hbm.at[idx_v], rows_v, sem).wait()
        for q in range(ACHUNK // 16):
            sv = idx_v[pl.ds(q * 16, 16)]
            dv = dstc_v[pl.ds(q * 16, 16)]
            di_s = plsc.load_gather(dinv_v, [sv])
            di_d = plsc.load_gather(dinv_v, [dv])
            nrm_v[pl.ds(q * 16, 16)] = di_s * di_d
        k1 = jnp.minimum(cnt - g * ACHUNK, ACHUNK)

        def edge_body(k, _):
            dval = dstc_v[pl.ds(k, 16)][0]
            nrm = nrm_v[pl.ds(k, 16)][0]
            r = dval - n0
            for j in range(8):
                row = rows_v[k, pl.ds(j * 16, 16)]
                tmp = row * nrm
                out_v[r, pl.ds(j * 16, 16)] = out_v[r, pl.ds(j * 16, 16)] + tmp
            return 0

        lax.fori_loop(0, k1, edge_body, 0)
        return 0

    lax.fori_loop(0, nch, chunk_body, 0)

    def self_chunk(q, _):
        rbase = q * ACHUNK
        sz = jnp.minimum(n1 - n0 - rbase, ACHUNK)
        pltpu.sync_copy(hw_hbm.at[pl.ds(n0 + rbase, ACHUNK)], rows_v)

        def srow(r, _):
            dself = dinv_v[pl.ds(n0 + rbase + r, 16)][0]
            nself = dself * dself
            for j in range(8):
                row = rows_v[r, pl.ds(j * 16, 16)]
                tmp = row * nself
                out_v[rbase + r, pl.ds(j * 16, 16)] = (
                    out_v[rbase + r, pl.ds(j * 16, 16)] + tmp)
            return 0

        lax.fori_loop(0, sz, srow, 0)
        return 0

    lax.fori_loop(0, (_NB_MAX + ACHUNK - 1) // ACHUNK, self_chunk, 0)

    pltpu.sync_copy(out_v.at[pl.ds(0, 312)], out_hbm.at[pl.ds(n0, 312)])

    @pl.when(n1 - n0 == 320)
    def _():
        pltpu.sync_copy(out_v.at[pl.ds(312, 8)], out_hbm.at[pl.ds(n0 + 312, 8)])


_agg_call = functools.partial(
    pl.kernel,
    out_type=jax.ShapeDtypeStruct((N, D), jnp.float32),
    mesh=_MESH,
    compiler_params=pltpu.CompilerParams(needs_layout_passes=False),
    scratch_types=[
        pltpu.VMEM((ACHUNK,), jnp.int32),
        pltpu.VMEM((ACHUNK + 16,), jnp.int32),
        pltpu.VMEM((ACHUNK + 16,), jnp.float32),
        pltpu.VMEM((N + 16,), jnp.float32),
        pltpu.VMEM((ACHUNK, D), jnp.float32),
        pltpu.VMEM((_NB_MAX, D), jnp.float32),
        pltpu.VMEM((16,), jnp.int32),
        pltpu.SemaphoreType.DMA,
    ],
)(_agg_body)


def kernel(x, edge_index, W1, b1, Wp, bp):
    src, dst = edge_index[0], edge_index[1]
    bounds = jnp.asarray(_BOUNDS)
    tile_of = jnp.searchsorted(bounds, dst, side="right") - 1
    onehot = (tile_of[:, None] == jnp.arange(NCHIP)[None, :]).astype(jnp.int32)
    cum = jnp.cumsum(onehot, axis=0)
    counts = cum[-1]
    rank_within = jnp.take_along_axis(cum, tile_of[:, None], axis=1)[:, 0] - 1
    pos = tile_of * CAP + rank_within
    tsrc = jnp.zeros((NCHIP * CAP,), jnp.int32).at[pos].add(
        src, unique_indices=True)
    tdst = jnp.zeros((NCHIP * CAP,), jnp.int32).at[pos].add(
        dst, unique_indices=True)
    tcnt = jnp.pad(counts.astype(jnp.int32)[:, None],
                   ((0, 0), (0, 15))).reshape(-1)

    loop = jnp.arange(N, dtype=src.dtype)
    d = jnp.concatenate([dst, loop])
    deg = jnp.zeros((N,), x.dtype).at[d].add(1.0)
    dinv = jnp.where(deg > 0, 1.0 / jnp.sqrt(deg), 0.0)

    pad_rows = jnp.zeros((ACHUNK, D), jnp.float32)

    def conv(h):
        hw = _matmul(h, W1)
        hwp = jnp.concatenate([hw, pad_rows])
        return _agg_call(tsrc, tdst, tcnt, dinv, hwp) + b1

    h = jax.nn.relu(conv(x))
    h1 = conv(h)
    g = h1 @ Wp + bp
    order = jnp.argsort(g[:, 0])
    gs = g[order]
    h1p = h1[order]
    scaled = pl.pallas_call(
        _scale_kernel,
        out_shape=jax.ShapeDtypeStruct((N, D), x.dtype),
        grid=(N // _ROWS,),
        in_specs=[
            pl.BlockSpec((_ROWS, 1), lambda i: (i, 0)),
            pl.BlockSpec((_ROWS, D), lambda i: (i, 0)),
        ],
        out_specs=pl.BlockSpec((_ROWS, D), lambda i: (i, 0)),
    )(gs, h1p)
    return jnp.transpose(scaled, (1, 0))[None, :, :]
